# Initial kernel scaffold; baseline (speedup 1.0000x reference)
import jax
import jax.numpy as jnp
from jax.experimental import pallas as pl


def kernel(x, edge_index, W1, b1, W2, b2, W3, b3):
    raise NotImplementedError("write your pallas kernel here")


# speedup vs baseline: 9.0973x; 9.0973x over previous
import functools

import jax
import jax.numpy as jnp
from jax import lax
from jax.experimental import pallas as pl
from jax.experimental.pallas import tpu as pltpu
from jax.experimental.pallas import tpu_sc as plsc

N = 10000
E = 320000
D = 128
DC = 48
NC = 2
NS = 16
NW = NC * NS
CH = 80
G = E // NW // CH
RPS = N // NS
ZR = 125
BLK = 2000

assert G % 2 == 1 and E == NW * G * CH and RPS % ZR == 0

_MESH = plsc.VectorSubcoreMesh(core_axis_name="c", subcore_axis_name="s")
_SC_PARAMS = pltpu.CompilerParams(use_tc_tiling_on_sc=False)


def _make_sc_aggregate(width):
    out_type = [jax.ShapeDtypeStruct((NW, RPS, width), jnp.float32)]
    scratch = [
        pltpu.VMEM_SHARED((N, width), jnp.float32),
        pltpu.VMEM((G, CH), jnp.int32),
        pltpu.VMEM((G, CH), jnp.int32),
        pltpu.VMEM((CH, width), jnp.float32),
        pltpu.VMEM((CH, width), jnp.float32),
        pltpu.SemaphoreType.DMA,
        pltpu.SemaphoreType.DMA,
    ]

    def body(y, src2d, dst2d, out, acc, src_idx, dst_idx, rows_a, rows_b,
             sem_a, sem_b):
        c = lax.axis_index("c")
        s = lax.axis_index("s")
        wid = c * NS + s

        pltpu.sync_copy(src2d.at[wid], src_idx)
        pltpu.sync_copy(dst2d.at[wid], dst_idx)

        z16 = jnp.zeros((16,), jnp.float32)

        def zero_rows(i, carry):
            for t in range(width // 16):
                rows_a[i, pl.ds(t * 16, 16)] = z16
            return carry

        lax.fori_loop(0, CH, zero_rows, 0)
        for k in range(RPS // CH):
            pltpu.sync_copy(rows_a, acc.at[pl.ds(s * RPS + k * CH, CH)])
        _rem = RPS % CH
        if _rem:
            pltpu.sync_copy(rows_a.at[pl.ds(0, _rem)],
                            acc.at[pl.ds(s * RPS + (RPS // CH) * CH, _rem)])

        plsc.subcore_barrier()

        pltpu.async_copy(y.at[src_idx.at[0]], rows_a, sem_a)

        def pair(jj, carry):
            j0 = 2 * jj
            pltpu.make_async_copy(y.at[src_idx.at[j0]], rows_a, sem_a).wait()
            pltpu.async_copy(y.at[src_idx.at[j0 + 1]], rows_b, sem_b)
            pltpu.sync_copy(rows_a, acc.at[dst_idx.at[j0]], add=True)
            pltpu.make_async_copy(y.at[src_idx.at[j0 + 1]], rows_b, sem_b).wait()
            pltpu.async_copy(y.at[src_idx.at[j0 + 2]], rows_a, sem_a)
            pltpu.sync_copy(rows_b, acc.at[dst_idx.at[j0 + 1]], add=True)
            return carry

        lax.fori_loop(0, (G - 1) // 2, pair, 0)
        pltpu.make_async_copy(y.at[src_idx.at[G - 1]], rows_a, sem_a).wait()
        pltpu.sync_copy(rows_a, acc.at[dst_idx.at[G - 1]], add=True)

        plsc.subcore_barrier()

        pltpu.sync_copy(acc.at[pl.ds(s * RPS, RPS)], out.at[wid])

    return pl.kernel(body, out_type=out_type, mesh=_MESH,
                     scratch_types=scratch, compiler_params=_SC_PARAMS)


def _make_sc_degrees():
    out_type = [jax.ShapeDtypeStruct((NW, RPS, 32), jnp.float32)]
    scratch = [
        pltpu.VMEM_SHARED((N, 32), jnp.float32),
        pltpu.VMEM((G, CH), jnp.int32),
        pltpu.VMEM((G, CH), jnp.int32),
        pltpu.VMEM((CH, 32), jnp.float32),
        pltpu.VMEM((CH, 32), jnp.float32),
        pltpu.VMEM((RPS, 32), jnp.float32),
    ]

    def body(src2d, dst2d, out, hist, src_idx, dst_idx, ones_s, ones_d, zh):
        c = lax.axis_index("c")
        s = lax.axis_index("s")
        wid = c * NS + s

        pltpu.sync_copy(src2d.at[wid], src_idx)
        pltpu.sync_copy(dst2d.at[wid], dst_idx)

        z16 = jnp.zeros((16,), jnp.float32)
        one16 = jnp.ones((16,), jnp.float32)

        def zero_zh(i, carry):
            zh[i, pl.ds(0, 16)] = z16
            zh[i, pl.ds(16, 16)] = z16
            return carry

        lax.fori_loop(0, RPS, zero_zh, 0)

        def fill_ones(i, carry):
            ones_s[i, pl.ds(0, 16)] = one16
            ones_s[i, pl.ds(16, 16)] = z16
            ones_d[i, pl.ds(0, 16)] = z16
            ones_d[i, pl.ds(16, 16)] = one16
            return carry

        lax.fori_loop(0, CH, fill_ones, 0)

        pltpu.sync_copy(zh, hist.at[pl.ds(s * RPS, RPS)])
        plsc.subcore_barrier()

        def chunk(j, carry):
            pltpu.sync_copy(ones_s, hist.at[src_idx.at[j]], add=True)
            pltpu.sync_copy(ones_d, hist.at[dst_idx.at[j]], add=True)
            return carry

        lax.fori_loop(0, G, chunk, 0)
        plsc.subcore_barrier()
        pltpu.sync_copy(hist.at[pl.ds(s * RPS, RPS)], out.at[wid])

    return pl.kernel(body, out_type=out_type, mesh=_MESH,
                     scratch_types=scratch, compiler_params=_SC_PARAMS)


_agg_l1 = _make_sc_aggregate(D)
_agg_l2 = _make_sc_aggregate(D)
_agg_l3 = _make_sc_aggregate(DC)
_degrees = _make_sc_degrees()

_NB = N // BLK


def _blk(i):
    return (i, 0)


def _blk_hi(i):
    return (_NB + i, 0)


def _fixed(i):
    return (0, 0)


def _tc1_body(x_ref, w_ref, o_ref):
    o_ref[:, :] = jnp.dot(x_ref[:, :], w_ref[:, :],
                          preferred_element_type=jnp.float32)


def _tc2_body(a_ref, b_ref, ha, hb, b1r, w2, y2o, nso, ndo):
    h1 = jnp.maximum(a_ref[:, :] + b_ref[:, :] + b1r[:, :], 0.0)
    dout = ha[:, 0:1] + hb[:, 0:1]
    din = ha[:, 16:17] + hb[:, 16:17]
    ns = jnp.where(dout > 0, lax.rsqrt(dout), 0.0)
    nd = jnp.where(din > 0, lax.rsqrt(din), 0.0)
    y2o[:, :] = jnp.dot(h1, w2[:, :], preferred_element_type=jnp.float32) * ns
    nso[:, :] = ns
    ndo[:, :] = nd


def _tc3_body(a_ref, b_ref, nd, ns, b2r, w3, y3o):
    h2 = jnp.maximum((a_ref[:, :] + b_ref[:, :]) * nd[:, :] + b2r[:, :], 0.0)
    y3o[:, :] = jnp.dot(h2, w3[:, :], preferred_element_type=jnp.float32) * ns[:, :]


def _tc4_body(a_ref, b_ref, nd, b3r, o_ref):
    o_ref[:, :] = (a_ref[:, :] + b_ref[:, :]) * nd[:, :] + b3r[:, :]


def _spec(w, index_map=_blk, rows=BLK):
    return pl.BlockSpec((rows, w), index_map)


def kernel(x, edge_index, W1, b1, W2, b2, W3, b3):
    src3d = edge_index[0].reshape(NW, G, CH)
    dst3d = edge_index[1].reshape(NW, G, CH)
    b1r = b1.reshape(1, D)
    b2r = b2.reshape(1, D)
    w3p = jnp.pad(W3, ((0, 0), (0, DC - W3.shape[1])))
    b3r = jnp.pad(b3, (0, DC - b3.shape[0])).reshape(1, DC)

    y1 = pl.pallas_call(
        _tc1_body, grid=(_NB,),
        in_specs=[_spec(D), pl.BlockSpec((D, D), _fixed)],
        out_specs=_spec(D),
        out_shape=jax.ShapeDtypeStruct((N, D), jnp.float32),
    )(x, W1)

    hist, = _degrees(src3d, dst3d)
    hist = hist.reshape(NC * N, 32)
    agg1, = _agg_l1(y1, src3d, dst3d)
    agg1 = agg1.reshape(NC * N, D)

    y2, nsrc, ndst = pl.pallas_call(
        _tc2_body, grid=(_NB,),
        in_specs=[_spec(D), _spec(D, _blk_hi),
                  _spec(32), _spec(32, _blk_hi),
                  pl.BlockSpec((1, D), _fixed), pl.BlockSpec((D, D), _fixed)],
        out_specs=[_spec(D), _spec(1), _spec(1)],
        out_shape=[jax.ShapeDtypeStruct((N, D), jnp.float32),
                   jax.ShapeDtypeStruct((N, 1), jnp.float32),
                   jax.ShapeDtypeStruct((N, 1), jnp.float32)],
    )(agg1, agg1, hist, hist, b1r, W2)

    agg2, = _agg_l2(y2, src3d, dst3d)
    agg2 = agg2.reshape(NC * N, D)

    y3 = pl.pallas_call(
        _tc3_body, grid=(_NB,),
        in_specs=[_spec(D), _spec(D, _blk_hi), _spec(1), _spec(1),
                  pl.BlockSpec((1, D), _fixed), pl.BlockSpec((D, DC), _fixed)],
        out_specs=_spec(DC),
        out_shape=jax.ShapeDtypeStruct((N, DC), jnp.float32),
    )(agg2, agg2, ndst, nsrc, b2r, w3p)

    agg3, = _agg_l3(y3, src3d, dst3d)
    agg3 = agg3.reshape(NC * N, DC)

    out = pl.pallas_call(
        _tc4_body, grid=(_NB,),
        in_specs=[_spec(DC), _spec(DC, _blk_hi), _spec(1),
                  pl.BlockSpec((1, DC), _fixed)],
        out_specs=_spec(DC),
        out_shape=jax.ShapeDtypeStruct((N, DC), jnp.float32),
    )(agg3, agg3, ndst, b3r)

    return out[:, :W3.shape[1]]


# speedup vs baseline: 11.8151x; 1.2987x over previous
import functools

import jax
import jax.numpy as jnp
from jax import lax
from jax.experimental import pallas as pl
from jax.experimental.pallas import tpu as pltpu
from jax.experimental.pallas import tpu_sc as plsc

N = 10000
E = 320000
D = 128
DC = 48
NC = 2
NS = 16
NW = NC * NS
CH = 80
G = E // NW // CH
RPS = N // NS
ZR = 125
BLK = 2000

assert G % 2 == 1 and E == NW * G * CH and RPS % ZR == 0

_MESH = plsc.VectorSubcoreMesh(core_axis_name="c", subcore_axis_name="s")
_SC_PARAMS = pltpu.CompilerParams(use_tc_tiling_on_sc=False)


def _make_sc_aggregate(width):
    out_type = [jax.ShapeDtypeStruct((NW, RPS, width), jnp.float32)]
    scratch = [
        pltpu.VMEM_SHARED((N, width), jnp.float32),
        pltpu.VMEM((G, CH), jnp.int32),
        pltpu.VMEM((G, CH), jnp.int32),
        pltpu.VMEM((CH, width), jnp.float32),
        pltpu.VMEM((CH, width), jnp.float32),
        pltpu.VMEM((CH, width), jnp.float32),
        pltpu.SemaphoreType.DMA,
        pltpu.SemaphoreType.DMA,
        pltpu.SemaphoreType.DMA,
        pltpu.SemaphoreType.DMA,
        pltpu.SemaphoreType.DMA,
        pltpu.SemaphoreType.DMA,
    ]

    def body(y, src2d, dst2d, out, acc, src_idx, dst_idx, rows_a, rows_b,
             rows_c, gsem_a, gsem_b, gsem_c, ssem_a, ssem_b, ssem_c):
        c = lax.axis_index("c")
        s = lax.axis_index("s")
        wid = c * NS + s

        pltpu.sync_copy(src2d.at[wid], src_idx)
        pltpu.sync_copy(dst2d.at[wid], dst_idx)

        z16 = jnp.zeros((16,), jnp.float32)

        def zero_rows(i, carry):
            for t in range(width // 16):
                rows_a[i, pl.ds(t * 16, 16)] = z16
            return carry

        lax.fori_loop(0, CH, zero_rows, 0)
        for k in range(RPS // CH):
            pltpu.sync_copy(rows_a, acc.at[pl.ds(s * RPS + k * CH, CH)])
        _rem = RPS % CH
        if _rem:
            pltpu.sync_copy(rows_a.at[pl.ds(0, _rem)],
                            acc.at[pl.ds(s * RPS + (RPS // CH) * CH, _rem)])

        plsc.subcore_barrier()

        bufs = (rows_a, rows_b, rows_c)
        gsems = (gsem_a, gsem_b, gsem_c)
        ssems = (ssem_a, ssem_b, ssem_c)

        def gath(j, b):
            pltpu.async_copy(y.at[src_idx.at[j]], bufs[b], gsems[b])

        def wait_gath(j, b):
            pltpu.make_async_copy(y.at[src_idx.at[j]], bufs[b],
                                  gsems[b]).wait()

        def scat(j, b):
            pltpu.async_copy(bufs[b], acc.at[dst_idx.at[j]], ssems[b],
                             add=True)

        def drain_scat(j, b):
            pltpu.make_async_copy(bufs[b], acc.at[dst_idx.at[j]],
                                  ssems[b]).wait()

        gath(0, 0)
        gath(1, 1)
        gath(2, 2)

        def trio(jj, carry):
            a = 3 * jj
            wait_gath(a, 0)
            scat(a, 0)
            wait_gath(a + 1, 1)
            scat(a + 1, 1)
            wait_gath(a + 2, 2)
            scat(a + 2, 2)
            drain_scat(a, 0)
            gath(a + 3, 0)
            drain_scat(a + 1, 1)
            gath(a + 4, 1)
            drain_scat(a + 2, 2)

            @pl.when(a + 5 < G)
            def _():
                gath(a + 5, 2)

            return carry

        assert G % 3 == 2
        lax.fori_loop(0, (G - 2) // 3, trio, 0)
        wait_gath(G - 2, 0)
        scat(G - 2, 0)
        wait_gath(G - 1, 1)
        scat(G - 1, 1)
        drain_scat(G - 2, 0)
        drain_scat(G - 1, 1)

        plsc.subcore_barrier()

        pltpu.sync_copy(acc.at[pl.ds(s * RPS, RPS)], out.at[wid])

    return pl.kernel(body, out_type=out_type, mesh=_MESH,
                     scratch_types=scratch, compiler_params=_SC_PARAMS)


def _make_sc_degrees():
    out_type = [jax.ShapeDtypeStruct((NW, RPS, 32), jnp.float32)]
    scratch = [
        pltpu.VMEM_SHARED((N, 32), jnp.float32),
        pltpu.VMEM((G, CH), jnp.int32),
        pltpu.VMEM((G, CH), jnp.int32),
        pltpu.VMEM((CH, 32), jnp.float32),
        pltpu.VMEM((CH, 32), jnp.float32),
        pltpu.VMEM((RPS, 32), jnp.float32),
        pltpu.SemaphoreType.DMA,
    ]

    def body(src2d, dst2d, out, hist, src_idx, dst_idx, ones_s, ones_d, zh,
             sem):
        c = lax.axis_index("c")
        s = lax.axis_index("s")
        wid = c * NS + s

        pltpu.sync_copy(src2d.at[wid], src_idx)
        pltpu.sync_copy(dst2d.at[wid], dst_idx)

        z16 = jnp.zeros((16,), jnp.float32)
        one16 = jnp.ones((16,), jnp.float32)

        def zero_zh(i, carry):
            zh[i, pl.ds(0, 16)] = z16
            zh[i, pl.ds(16, 16)] = z16
            return carry

        lax.fori_loop(0, RPS, zero_zh, 0)

        def fill_ones(i, carry):
            ones_s[i, pl.ds(0, 16)] = one16
            ones_s[i, pl.ds(16, 16)] = z16
            ones_d[i, pl.ds(0, 16)] = z16
            ones_d[i, pl.ds(16, 16)] = one16
            return carry

        lax.fori_loop(0, CH, fill_ones, 0)

        pltpu.sync_copy(zh, hist.at[pl.ds(s * RPS, RPS)])
        plsc.subcore_barrier()

        lag = 8

        def chunk(j, carry):
            pltpu.async_copy(ones_s, hist.at[src_idx.at[j]], sem, add=True)
            pltpu.async_copy(ones_d, hist.at[dst_idx.at[j]], sem, add=True)

            @pl.when(j >= lag)
            def _():
                pltpu.make_async_copy(ones_s, hist.at[src_idx.at[j]],
                                      sem).wait()
                pltpu.make_async_copy(ones_d, hist.at[dst_idx.at[j]],
                                      sem).wait()

            return carry

        lax.fori_loop(0, G, chunk, 0)
        for _ in range(lag):
            pltpu.make_async_copy(ones_s, hist.at[src_idx.at[0]], sem).wait()
            pltpu.make_async_copy(ones_d, hist.at[dst_idx.at[0]], sem).wait()
        plsc.subcore_barrier()
        pltpu.sync_copy(hist.at[pl.ds(s * RPS, RPS)], out.at[wid])

    return pl.kernel(body, out_type=out_type, mesh=_MESH,
                     scratch_types=scratch, compiler_params=_SC_PARAMS)


_agg_l1 = _make_sc_aggregate(D)
_agg_l2 = _make_sc_aggregate(D)
_agg_l3 = _make_sc_aggregate(DC)
_degrees = _make_sc_degrees()

_NB = N // BLK


def _blk(i):
    return (i, 0)


def _blk_hi(i):
    return (_NB + i, 0)


def _fixed(i):
    return (0, 0)


def _tc1_body(x_ref, w_ref, o_ref):
    o_ref[:, :] = jnp.dot(x_ref[:, :], w_ref[:, :],
                          preferred_element_type=jnp.float32)


def _tc2_body(a_ref, b_ref, ha, hb, b1r, w2, y2o, nso, ndo):
    h1 = jnp.maximum(a_ref[:, :] + b_ref[:, :] + b1r[:, :], 0.0)
    dout = ha[:, 0:1] + hb[:, 0:1]
    din = ha[:, 16:17] + hb[:, 16:17]
    ns = jnp.where(dout > 0, lax.rsqrt(dout), 0.0)
    nd = jnp.where(din > 0, lax.rsqrt(din), 0.0)
    y2o[:, :] = jnp.dot(h1, w2[:, :], preferred_element_type=jnp.float32) * ns
    nso[:, :] = ns
    ndo[:, :] = nd


def _tc3_body(a_ref, b_ref, nd, ns, b2r, w3, y3o):
    h2 = jnp.maximum((a_ref[:, :] + b_ref[:, :]) * nd[:, :] + b2r[:, :], 0.0)
    y3o[:, :] = jnp.dot(h2, w3[:, :], preferred_element_type=jnp.float32) * ns[:, :]


def _tc4_body(a_ref, b_ref, nd, b3r, o_ref):
    o_ref[:, :] = (a_ref[:, :] + b_ref[:, :]) * nd[:, :] + b3r[:, :]


def _spec(w, index_map=_blk, rows=BLK):
    return pl.BlockSpec((rows, w), index_map)


def kernel(x, edge_index, W1, b1, W2, b2, W3, b3):
    src3d = edge_index[0].reshape(NW, G, CH)
    dst3d = edge_index[1].reshape(NW, G, CH)
    b1r = b1.reshape(1, D)
    b2r = b2.reshape(1, D)
    w3p = jnp.pad(W3, ((0, 0), (0, DC - W3.shape[1])))
    b3r = jnp.pad(b3, (0, DC - b3.shape[0])).reshape(1, DC)

    y1 = pl.pallas_call(
        _tc1_body, grid=(_NB,),
        in_specs=[_spec(D), pl.BlockSpec((D, D), _fixed)],
        out_specs=_spec(D),
        out_shape=jax.ShapeDtypeStruct((N, D), jnp.float32),
    )(x, W1)

    hist, = _degrees(src3d, dst3d)
    hist = hist.reshape(NC * N, 32)
    agg1, = _agg_l1(y1, src3d, dst3d)
    agg1 = agg1.reshape(NC * N, D)

    y2, nsrc, ndst = pl.pallas_call(
        _tc2_body, grid=(_NB,),
        in_specs=[_spec(D), _spec(D, _blk_hi),
                  _spec(32), _spec(32, _blk_hi),
                  pl.BlockSpec((1, D), _fixed), pl.BlockSpec((D, D), _fixed)],
        out_specs=[_spec(D), _spec(1), _spec(1)],
        out_shape=[jax.ShapeDtypeStruct((N, D), jnp.float32),
                   jax.ShapeDtypeStruct((N, 1), jnp.float32),
                   jax.ShapeDtypeStruct((N, 1), jnp.float32)],
    )(agg1, agg1, hist, hist, b1r, W2)

    agg2, = _agg_l2(y2, src3d, dst3d)
    agg2 = agg2.reshape(NC * N, D)

    y3 = pl.pallas_call(
        _tc3_body, grid=(_NB,),
        in_specs=[_spec(D), _spec(D, _blk_hi), _spec(1), _spec(1),
                  pl.BlockSpec((1, D), _fixed), pl.BlockSpec((D, DC), _fixed)],
        out_specs=_spec(DC),
        out_shape=jax.ShapeDtypeStruct((N, DC), jnp.float32),
    )(agg2, agg2, ndst, nsrc, b2r, w3p)

    agg3, = _agg_l3(y3, src3d, dst3d)
    agg3 = agg3.reshape(NC * N, DC)

    out = pl.pallas_call(
        _tc4_body, grid=(_NB,),
        in_specs=[_spec(DC), _spec(DC, _blk_hi), _spec(1),
                  pl.BlockSpec((1, DC), _fixed)],
        out_specs=_spec(DC),
        out_shape=jax.ShapeDtypeStruct((N, DC), jnp.float32),
    )(agg3, agg3, ndst, b3r)

    return out[:, :W3.shape[1]]


# speedup vs baseline: 12.4780x; 1.0561x over previous
import functools

import jax
import jax.numpy as jnp
from jax import lax
from jax.experimental import pallas as pl
from jax.experimental.pallas import tpu as pltpu
from jax.experimental.pallas import tpu_sc as plsc

N = 10000
E = 320000
D = 128
DC = 48
NC = 2
NS = 16
NW = NC * NS
CH = 80
G = E // NW // CH
RPS = N // NS
ZR = 125
BLK = 2000

assert G % 2 == 1 and E == NW * G * CH and RPS % ZR == 0

_MESH = plsc.VectorSubcoreMesh(core_axis_name="c", subcore_axis_name="s")
_SC_PARAMS = pltpu.CompilerParams(use_tc_tiling_on_sc=False)


def _make_sc_aggregate(width, nbuf):
    out_type = [jax.ShapeDtypeStruct((NW, RPS, width), jnp.float32)]
    scratch = (
        [pltpu.VMEM_SHARED((N, width), jnp.float32)]
        + [pltpu.VMEM((G, CH), jnp.int32)] * 2
        + [pltpu.VMEM((CH, width), jnp.float32)] * nbuf
        + [pltpu.SemaphoreType.DMA] * (2 * nbuf)
    )

    def body(y, src2d, dst2d, out, acc, src_idx, dst_idx, *bufs_and_sems):
        bufs = bufs_and_sems[:nbuf]
        gsems = bufs_and_sems[nbuf:2 * nbuf]
        ssems = bufs_and_sems[2 * nbuf:3 * nbuf]
        rows_a = bufs[0]
        c = lax.axis_index("c")
        s = lax.axis_index("s")
        wid = c * NS + s

        pltpu.sync_copy(src2d.at[wid], src_idx)
        pltpu.sync_copy(dst2d.at[wid], dst_idx)

        z16 = jnp.zeros((16,), jnp.float32)

        def zero_rows(i, carry):
            for t in range(width // 16):
                rows_a[i, pl.ds(t * 16, 16)] = z16
            return carry

        lax.fori_loop(0, CH, zero_rows, 0)
        for k in range(RPS // CH):
            pltpu.sync_copy(rows_a, acc.at[pl.ds(s * RPS + k * CH, CH)])
        _rem = RPS % CH
        if _rem:
            pltpu.sync_copy(rows_a.at[pl.ds(0, _rem)],
                            acc.at[pl.ds(s * RPS + (RPS // CH) * CH, _rem)])

        plsc.subcore_barrier()

        def gath(j, b):
            pltpu.async_copy(y.at[src_idx.at[j]], bufs[b], gsems[b])

        def wait_gath(j, b):
            pltpu.make_async_copy(y.at[src_idx.at[j]], bufs[b],
                                  gsems[b]).wait()

        def scat(j, b):
            pltpu.async_copy(bufs[b], acc.at[dst_idx.at[j]], ssems[b],
                             add=True)

        def drain_scat(j, b):
            pltpu.make_async_copy(bufs[b], acc.at[dst_idx.at[j]],
                                  ssems[b]).wait()

        for b in range(nbuf):
            gath(b, b)

        def ring(ii, carry):
            a = nbuf * ii
            for b in range(nbuf):
                wait_gath(a + b, b)
                scat(a + b, b)
            for b in range(nbuf):
                drain_scat(a + b, b)
                nxt = a + nbuf + b

                @pl.when(nxt < G)
                def _(nxt=nxt, b=b):
                    gath(nxt, b)

            return carry

        lax.fori_loop(0, G // nbuf, ring, 0)
        _base = (G // nbuf) * nbuf
        for b in range(G % nbuf):
            wait_gath(_base + b, b)
            scat(_base + b, b)
        for b in range(G % nbuf):
            drain_scat(_base + b, b)

        plsc.subcore_barrier()

        pltpu.sync_copy(acc.at[pl.ds(s * RPS, RPS)], out.at[wid])

    return pl.kernel(body, out_type=out_type, mesh=_MESH,
                     scratch_types=scratch, compiler_params=_SC_PARAMS)


def _make_sc_degrees():
    out_type = [jax.ShapeDtypeStruct((NW, RPS, 16), jnp.float32)] * 2
    scratch = [
        pltpu.VMEM_SHARED((N, 16), jnp.float32),
        pltpu.VMEM_SHARED((N, 16), jnp.float32),
        pltpu.VMEM((G, CH), jnp.int32),
        pltpu.VMEM((G, CH), jnp.int32),
        pltpu.VMEM((CH, 16), jnp.float32),
        pltpu.VMEM((RPS, 16), jnp.float32),
        pltpu.SemaphoreType.DMA,
    ]

    def body(src2d, dst2d, hs_out, hd_out, hsrc, hdst, src_idx, dst_idx,
             ones, zh, sem):
        c = lax.axis_index("c")
        s = lax.axis_index("s")
        wid = c * NS + s

        pltpu.sync_copy(src2d.at[wid], src_idx)
        pltpu.sync_copy(dst2d.at[wid], dst_idx)

        z16 = jnp.zeros((16,), jnp.float32)
        one16 = jnp.ones((16,), jnp.float32)

        def zero_zh(i, carry):
            zh[i, :] = z16
            return carry

        lax.fori_loop(0, RPS, zero_zh, 0)

        def fill_ones(i, carry):
            ones[i, :] = one16
            return carry

        lax.fori_loop(0, CH, fill_ones, 0)

        pltpu.sync_copy(zh, hsrc.at[pl.ds(s * RPS, RPS)])
        pltpu.sync_copy(zh, hdst.at[pl.ds(s * RPS, RPS)])
        plsc.subcore_barrier()

        lag = 8

        def chunk(j, carry):
            pltpu.async_copy(ones, hsrc.at[src_idx.at[j]], sem, add=True)
            pltpu.async_copy(ones, hdst.at[dst_idx.at[j]], sem, add=True)

            @pl.when(j >= lag)
            def _():
                pltpu.make_async_copy(ones, hsrc.at[src_idx.at[j]],
                                      sem).wait()
                pltpu.make_async_copy(ones, hdst.at[dst_idx.at[j]],
                                      sem).wait()

            return carry

        lax.fori_loop(0, G, chunk, 0)
        for _ in range(lag):
            pltpu.make_async_copy(ones, hsrc.at[src_idx.at[0]], sem).wait()
            pltpu.make_async_copy(ones, hdst.at[dst_idx.at[0]], sem).wait()
        plsc.subcore_barrier()
        pltpu.sync_copy(hsrc.at[pl.ds(s * RPS, RPS)], hs_out.at[wid])
        pltpu.sync_copy(hdst.at[pl.ds(s * RPS, RPS)], hd_out.at[wid])

    return pl.kernel(body, out_type=out_type, mesh=_MESH,
                     scratch_types=scratch, compiler_params=_SC_PARAMS)


_agg_l1 = _make_sc_aggregate(D, nbuf=3)
_agg_l2 = _make_sc_aggregate(D, nbuf=3)
_agg_l3 = _make_sc_aggregate(DC, nbuf=5)
_degrees = _make_sc_degrees()

_NB = N // BLK


def _blk(i):
    return (i, 0)


def _blk_hi(i):
    return (_NB + i, 0)


def _fixed(i):
    return (0, 0)


def _tc1_body(x_ref, w_ref, o_ref):
    o_ref[:, :] = jnp.dot(x_ref[:, :], w_ref[:, :],
                          preferred_element_type=jnp.float32)


def _tc2_body(a_ref, b_ref, hsa, hsb, hda, hdb, b1r, w2, y2o, nso, ndo):
    h1 = jnp.maximum(a_ref[:, :] + b_ref[:, :] + b1r[:, :], 0.0)
    dout = hsa[:, 0:1] + hsb[:, 0:1]
    din = hda[:, 0:1] + hdb[:, 0:1]
    ns = jnp.where(dout > 0, lax.rsqrt(dout), 0.0)
    nd = jnp.where(din > 0, lax.rsqrt(din), 0.0)
    y2o[:, :] = jnp.dot(h1, w2[:, :], preferred_element_type=jnp.float32) * ns
    nso[:, :] = ns
    ndo[:, :] = nd


def _tc3_body(a_ref, b_ref, nd, ns, b2r, w3, y3o):
    h2 = jnp.maximum((a_ref[:, :] + b_ref[:, :]) * nd[:, :] + b2r[:, :], 0.0)
    y3o[:, :] = jnp.dot(h2, w3[:, :], preferred_element_type=jnp.float32) * ns[:, :]


def _tc4_body(a_ref, b_ref, nd, b3r, o_ref):
    o_ref[:, :] = (a_ref[:, :] + b_ref[:, :]) * nd[:, :] + b3r[:, :]


def _spec(w, index_map=_blk, rows=BLK):
    return pl.BlockSpec((rows, w), index_map)


def kernel(x, edge_index, W1, b1, W2, b2, W3, b3):
    src3d = edge_index[0].reshape(NW, G, CH)
    dst3d = edge_index[1].reshape(NW, G, CH)
    b1r = b1.reshape(1, D)
    b2r = b2.reshape(1, D)
    w3p = jnp.pad(W3, ((0, 0), (0, DC - W3.shape[1])))
    b3r = jnp.pad(b3, (0, DC - b3.shape[0])).reshape(1, DC)

    y1 = pl.pallas_call(
        _tc1_body, grid=(_NB,),
        in_specs=[_spec(D), pl.BlockSpec((D, D), _fixed)],
        out_specs=_spec(D),
        out_shape=jax.ShapeDtypeStruct((N, D), jnp.float32),
    )(x, W1)

    hs, hd = _degrees(src3d, dst3d)
    hs = hs.reshape(NC * N, 16)
    hd = hd.reshape(NC * N, 16)
    agg1, = _agg_l1(y1, src3d, dst3d)
    agg1 = agg1.reshape(NC * N, D)

    y2, nsrc, ndst = pl.pallas_call(
        _tc2_body, grid=(_NB,),
        in_specs=[_spec(D), _spec(D, _blk_hi),
                  _spec(16), _spec(16, _blk_hi),
                  _spec(16), _spec(16, _blk_hi),
                  pl.BlockSpec((1, D), _fixed), pl.BlockSpec((D, D), _fixed)],
        out_specs=[_spec(D), _spec(1), _spec(1)],
        out_shape=[jax.ShapeDtypeStruct((N, D), jnp.float32),
                   jax.ShapeDtypeStruct((N, 1), jnp.float32),
                   jax.ShapeDtypeStruct((N, 1), jnp.float32)],
    )(agg1, agg1, hs, hs, hd, hd, b1r, W2)

    agg2, = _agg_l2(y2, src3d, dst3d)
    agg2 = agg2.reshape(NC * N, D)

    y3 = pl.pallas_call(
        _tc3_body, grid=(_NB,),
        in_specs=[_spec(D), _spec(D, _blk_hi), _spec(1), _spec(1),
                  pl.BlockSpec((1, D), _fixed), pl.BlockSpec((D, DC), _fixed)],
        out_specs=_spec(DC),
        out_shape=jax.ShapeDtypeStruct((N, DC), jnp.float32),
    )(agg2, agg2, ndst, nsrc, b2r, w3p)

    agg3, = _agg_l3(y3, src3d, dst3d)
    agg3 = agg3.reshape(NC * N, DC)

    out = pl.pallas_call(
        _tc4_body, grid=(_NB,),
        in_specs=[_spec(DC), _spec(DC, _blk_hi), _spec(1),
                  pl.BlockSpec((1, DC), _fixed)],
        out_specs=_spec(DC),
        out_shape=jax.ShapeDtypeStruct((N, DC), jnp.float32),
    )(agg3, agg3, ndst, b3r)

    return out[:, :W3.shape[1]]
